# Initial kernel scaffold; baseline (speedup 1.0000x reference)
import jax
import jax.numpy as jnp
from jax.experimental import pallas as pl


def kernel(logits, logits1, p0first, p0sec, orixyz):
    raise NotImplementedError("write your pallas kernel here")


# speedup vs baseline: 6.0711x; 6.0711x over previous
import functools

import jax
import jax.numpy as jnp
from jax.experimental import pallas as pl

BS = 4
NSAMPLE = 32
EPS_TCR = 0.2


def _stats_kernel(l_ref, l1_ref, cos_ref, rsq0_ref, rsq1_ref):
    b_i = pl.program_id(0)
    n_i = pl.program_id(1)
    x = l_ref[0]
    y = l1_ref[0]
    dot = jnp.sum(x * y, axis=-1, keepdims=True)
    rsq0 = jnp.sum(x * x, axis=-1, keepdims=True)
    rsq1 = jnp.sum(y * y, axis=-1, keepdims=True)
    nrm = jnp.maximum(jnp.sqrt(rsq0) * jnp.sqrt(rsq1), 1e-8)
    part = jnp.sum(dot / nrm)
    rsq0_ref[0] = rsq0
    rsq1_ref[0] = rsq1

    @pl.when(jnp.logical_and(b_i == 0, n_i == 0))
    def _():
        cos_ref[...] = jnp.reshape(part, (1, 1, 1))

    @pl.when(jnp.logical_not(jnp.logical_and(b_i == 0, n_i == 0)))
    def _():
        cos_ref[...] += jnp.reshape(part, (1, 1, 1))


def _run_stats(logits, logits1):
    b, n, c = logits.shape
    nb = 1024
    grid = (b, n // nb)
    cos_sum, rsq0, rsq1 = pl.pallas_call(
        _stats_kernel,
        grid=grid,
        in_specs=[
            pl.BlockSpec((1, nb, c), lambda bi, ni: (bi, ni, 0)),
            pl.BlockSpec((1, nb, c), lambda bi, ni: (bi, ni, 0)),
        ],
        out_specs=[
            pl.BlockSpec((1, 1, 1), lambda bi, ni: (0, 0, 0)),
            pl.BlockSpec((1, nb, 1), lambda bi, ni: (bi, ni, 0)),
            pl.BlockSpec((1, nb, 1), lambda bi, ni: (bi, ni, 0)),
        ],
        out_shape=[
            jax.ShapeDtypeStruct((1, 1, 1), jnp.float32),
            jax.ShapeDtypeStruct((b, n, 1), jnp.float32),
            jax.ShapeDtypeStruct((b, n, 1), jnp.float32),
        ],
    )(logits, logits1)
    return cos_sum[0, 0, 0], rsq0, rsq1


def _fps_kernel(xs_ref, ys_ref, zs_ref, idx_ref, *, npoint):
    xs = xs_ref[...]
    ys = ys_ref[...]
    zs = zs_ref[...]
    r, n = xs.shape
    fiota = jax.lax.broadcasted_iota(jnp.int32, (r, n), 1).astype(jnp.float32)
    iota_out = jax.lax.broadcasted_iota(jnp.int32, (r, npoint), 1)

    def step(i, carry):
        dists, far, idxs = carry
        m_out = (iota_out == i).astype(jnp.float32)
        idxs = idxs * (1.0 - m_out) + far * m_out
        cmask = fiota == far
        cx = jnp.sum(jnp.where(cmask, xs, 0.0), axis=1, keepdims=True)
        cy = jnp.sum(jnp.where(cmask, ys, 0.0), axis=1, keepdims=True)
        cz = jnp.sum(jnp.where(cmask, zs, 0.0), axis=1, keepdims=True)
        dx = xs - cx
        dy = ys - cy
        dz = zs - cz
        d = (dx * dx + dy * dy) + dz * dz
        dists = jnp.minimum(dists, d)
        mx = jnp.max(dists, axis=1, keepdims=True)
        far = jnp.min(jnp.where(dists == mx, fiota, float(n)), axis=1,
                      keepdims=True)
        return dists, far, idxs

    init = (xs * 0.0 + 1e10,
            jnp.min(xs * 0.0, axis=1, keepdims=True),
            xs[:, :npoint] * 0.0)
    _, _, idxs = jax.lax.fori_loop(0, npoint, step, init)
    idx_ref[...] = idxs


def _run_fps(xs, ys, zs, npoint):
    r, n = xs.shape
    return pl.pallas_call(
        functools.partial(_fps_kernel, npoint=npoint),
        out_shape=jax.ShapeDtypeStruct((r, npoint), jnp.float32),
    )(xs, ys, zs)


def _group_kernel(x_ref, xs_ref, ys_ref, zs_ref, rsq_ref, idx_ref, out_ref,
                  *, k, scalar):
    q_i = pl.program_id(1)
    x = x_ref[0]
    xs = xs_ref[0]
    ys = ys_ref[0]
    zs = zs_ref[0]
    rsq = rsq_ref[0]
    idx = idx_ref[0]
    qb, _ = idx.shape
    n = xs.shape[1]
    fiota = jax.lax.broadcasted_iota(jnp.int32, (qb, n), 1).astype(jnp.float32)

    sel_c = (idx == fiota)
    qx = jnp.sum(jnp.where(sel_c, xs, 0.0), axis=1, keepdims=True)
    qy = jnp.sum(jnp.where(sel_c, ys, 0.0), axis=1, keepdims=True)
    qz = jnp.sum(jnp.where(sel_c, zs, 0.0), axis=1, keepdims=True)
    qsq = (qx * qx + qy * qy) + qz * qz
    ssq = (xs * xs + ys * ys) + zs * zs
    cross = (qx * xs + qy * ys) + qz * zs
    d2 = (qsq + ssq) - 2.0 * cross

    def pick(i, carry):
        d2c, a = carry
        m = jnp.min(d2c, axis=1, keepdims=True)
        am = jnp.min(jnp.where(d2c <= m, fiota, float(n)), axis=1,
                     keepdims=True)
        sel = (fiota == am)
        a = a + jnp.where(sel, 1.0, 0.0)
        d2c = jnp.where(sel, 1e30, d2c)
        return d2c, a

    a0 = d2 * 0.0
    _, a = jax.lax.fori_loop(0, k, pick, (d2, a0))
    a = a + jnp.where(sel_c, 1.0, 0.0)

    recip = jnp.where(rsq > 0.0, jax.lax.rsqrt(rsq), 0.0)
    t_row = -0.5 * jnp.log1p(scalar * rsq)
    z = jnp.dot(a, x, preferred_element_type=jnp.float32)
    yw = jnp.dot(a * recip, x, preferred_element_type=jnp.float32)
    tsum = jnp.sum(a * t_row, axis=1, keepdims=True)

    yz = jnp.sum(yw * z, axis=1, keepdims=True)
    zn = jnp.sqrt(jnp.sum(z * z, axis=1, keepdims=True))
    cs_sum = yz / jnp.maximum(zn, 1e-30)
    npatch = float(k + 1)
    term = 200.0 * (-(cs_sum / npatch)) + tsum / npatch
    part = jnp.sum(term)

    @pl.when(q_i == 0)
    def _():
        out_ref[...] = jnp.reshape(part, (1, 1, 1))

    @pl.when(q_i != 0)
    def _():
        out_ref[...] += jnp.reshape(part, (1, 1, 1))


def _run_group(x, xs3, ys3, zs3, rsq3, idx3, k, qb):
    b, n, c = x.shape
    nq = idx3.shape[0] // b
    scalar = c / (BS * EPS_TCR)
    out = pl.pallas_call(
        functools.partial(_group_kernel, k=k, scalar=scalar),
        grid=(b, nq),
        in_specs=[
            pl.BlockSpec((1, n, c), lambda bi, qi: (bi, 0, 0)),
            pl.BlockSpec((1, 1, n), lambda bi, qi: (bi, 0, 0)),
            pl.BlockSpec((1, 1, n), lambda bi, qi: (bi, 0, 0)),
            pl.BlockSpec((1, 1, n), lambda bi, qi: (bi, 0, 0)),
            pl.BlockSpec((1, 1, n), lambda bi, qi: (bi, 0, 0)),
            pl.BlockSpec((1, qb, 1), lambda bi, qi, nq=nq: (bi * nq + qi, 0, 0)),
        ],
        out_specs=pl.BlockSpec((1, 1, 1), lambda bi, qi: (bi, 0, 0)),
        out_shape=jax.ShapeDtypeStruct((b, 1, 1), jnp.float32),
    )(x, xs3, ys3, zs3, rsq3, idx3)
    return out[:, 0, 0]


def kernel(logits, logits1, p0first, p0sec, orixyz):
    b, n, c = logits.shape
    downsample = n // 16
    qb = 128

    cos_sum, rsq0, rsq1 = _run_stats(logits, logits1)
    globalpointloss = -(cos_sum / (n * b))

    p0f = jax.lax.stop_gradient(p0first)
    p0s = jax.lax.stop_gradient(p0sec)
    xs = jnp.concatenate([p0f[:, :, 0], p0s[:, :, 0]], axis=0)
    ys = jnp.concatenate([p0f[:, :, 1], p0s[:, :, 1]], axis=0)
    zs = jnp.concatenate([p0f[:, :, 2], p0s[:, :, 2]], axis=0)

    idx_all = _run_fps(xs, ys, zs, downsample)
    nq = downsample // qb
    idx0 = idx_all[:b].reshape(b * nq, qb, 1)
    idx1 = idx_all[b:].reshape(b * nq, qb, 1)

    t0 = _run_group(logits, xs[:b, None, :], ys[:b, None, :], zs[:b, None, :],
                    rsq0.reshape(b, 1, n), idx0, NSAMPLE, qb)
    t1 = _run_group(logits1, xs[b:, None, :], ys[b:, None, :], zs[b:, None, :],
                    rsq1.reshape(b, 1, n), idx1, NSAMPLE, qb)

    g = jnp.float32(0.0)
    for index in range(b):
        g = (g + t0[index] + t1[index]) / downsample
    groupinnerloss = g / b
    return globalpointloss + groupinnerloss + groupinnerloss


# speedup vs baseline: 9.3264x; 1.5362x over previous
import functools

import jax
import jax.numpy as jnp
from jax.experimental import pallas as pl
from jax.experimental.pallas import tpu as pltpu

BS = 4
NSAMPLE = 32
EPS_TCR = 0.2


def _stats_kernel(l_ref, l1_ref, cos_ref, rsq0_ref, rsq1_ref):
    b_i = pl.program_id(0)
    n_i = pl.program_id(1)
    x = l_ref[0]
    y = l1_ref[0]
    dot = jnp.sum(x * y, axis=-1, keepdims=True)
    rsq0 = jnp.sum(x * x, axis=-1, keepdims=True)
    rsq1 = jnp.sum(y * y, axis=-1, keepdims=True)
    nrm = jnp.maximum(jnp.sqrt(rsq0) * jnp.sqrt(rsq1), 1e-8)
    part = jnp.sum(dot / nrm)
    rsq0_ref[0] = rsq0
    rsq1_ref[0] = rsq1

    @pl.when(jnp.logical_and(b_i == 0, n_i == 0))
    def _():
        cos_ref[...] = jnp.reshape(part, (1, 1, 1))

    @pl.when(jnp.logical_not(jnp.logical_and(b_i == 0, n_i == 0)))
    def _():
        cos_ref[...] += jnp.reshape(part, (1, 1, 1))


def _run_stats(logits, logits1):
    b, n, c = logits.shape
    nb = 1024
    grid = (b, n // nb)
    cos_sum, rsq0, rsq1 = pl.pallas_call(
        _stats_kernel,
        grid=grid,
        in_specs=[
            pl.BlockSpec((1, nb, c), lambda bi, ni: (bi, ni, 0)),
            pl.BlockSpec((1, nb, c), lambda bi, ni: (bi, ni, 0)),
        ],
        out_specs=[
            pl.BlockSpec((1, 1, 1), lambda bi, ni: (0, 0, 0)),
            pl.BlockSpec((1, nb, 1), lambda bi, ni: (bi, ni, 0)),
            pl.BlockSpec((1, nb, 1), lambda bi, ni: (bi, ni, 0)),
        ],
        out_shape=[
            jax.ShapeDtypeStruct((1, 1, 1), jnp.float32),
            jax.ShapeDtypeStruct((b, n, 1), jnp.float32),
            jax.ShapeDtypeStruct((b, n, 1), jnp.float32),
        ],
    )(logits, logits1)
    return cos_sum[0, 0, 0], rsq0, rsq1


def _fps_kernel(xs_ref, ys_ref, zs_ref, idx_ref, *, npoint):
    xs = xs_ref[...]
    ys = ys_ref[...]
    zs = zs_ref[...]
    r, n = xs.shape
    fiota = jax.lax.broadcasted_iota(jnp.int32, (r, n), 1).astype(jnp.float32)
    iota_out = jax.lax.broadcasted_iota(jnp.int32, (r, npoint), 1)

    def step(i, carry):
        dists, far, idxs = carry
        m_out = (iota_out == i).astype(jnp.float32)
        idxs = idxs * (1.0 - m_out) + far * m_out
        cmask = fiota == far
        cx = jnp.sum(jnp.where(cmask, xs, 0.0), axis=1, keepdims=True)
        cy = jnp.sum(jnp.where(cmask, ys, 0.0), axis=1, keepdims=True)
        cz = jnp.sum(jnp.where(cmask, zs, 0.0), axis=1, keepdims=True)
        dx = xs - cx
        dy = ys - cy
        dz = zs - cz
        d = (dx * dx + dy * dy) + dz * dz
        dists = jnp.minimum(dists, d)
        mx = jnp.max(dists, axis=1, keepdims=True)
        far = jnp.min(jnp.where(dists == mx, fiota, float(n)), axis=1,
                      keepdims=True)
        return dists, far, idxs

    init = (xs * 0.0 + 1e10,
            jnp.min(xs * 0.0, axis=1, keepdims=True),
            xs[:, :npoint] * 0.0)
    _, _, idxs = jax.lax.fori_loop(0, npoint, step, init)
    idx_ref[...] = idxs


def _run_fps(xs, ys, zs, npoint):
    r, n = xs.shape
    return pl.pallas_call(
        functools.partial(_fps_kernel, npoint=npoint),
        out_shape=jax.ShapeDtypeStruct((r, npoint), jnp.float32),
    )(xs, ys, zs)


def _group_kernel(x_ref, xs_ref, ys_ref, zs_ref, rsq_ref, idx_ref, out_ref,
                  *, k, scalar):
    q_i = pl.program_id(1)
    x = x_ref[0]
    xs = xs_ref[0]
    ys = ys_ref[0]
    zs = zs_ref[0]
    rsq = rsq_ref[0]
    idx = idx_ref[0]
    qb, _ = idx.shape
    n = xs.shape[1]
    fiota = jax.lax.broadcasted_iota(jnp.int32, (qb, n), 1).astype(jnp.float32)

    sel_c = (idx == fiota)
    qx = jnp.sum(jnp.where(sel_c, xs, 0.0), axis=1, keepdims=True)
    qy = jnp.sum(jnp.where(sel_c, ys, 0.0), axis=1, keepdims=True)
    qz = jnp.sum(jnp.where(sel_c, zs, 0.0), axis=1, keepdims=True)
    qsq = (qx * qx + qy * qy) + qz * qz
    ssq = (xs * xs + ys * ys) + zs * zs
    cross = (qx * xs + qy * ys) + qz * zs
    d2 = (qsq + ssq) - 2.0 * cross

    def pick(i, d2c):
        m = jnp.min(d2c, axis=1, keepdims=True)
        am = jnp.min(jnp.where(d2c <= m, fiota, float(n)), axis=1,
                     keepdims=True)
        return jnp.where(fiota == am, 1e30, d2c)

    d2f = jax.lax.fori_loop(0, k, pick, d2)
    a = jnp.where(d2f == 1e30, 1.0, 0.0)
    a = a + jnp.where(sel_c, 1.0, 0.0)

    recip = jnp.where(rsq > 0.0, jax.lax.rsqrt(rsq), 0.0)
    t_row = -0.5 * jnp.log1p(scalar * rsq)
    z = jnp.dot(a, x, preferred_element_type=jnp.float32)
    yw = jnp.dot(a * recip, x, preferred_element_type=jnp.float32)
    tsum = jnp.sum(a * t_row, axis=1, keepdims=True)

    yz = jnp.sum(yw * z, axis=1, keepdims=True)
    zn = jnp.sqrt(jnp.sum(z * z, axis=1, keepdims=True))
    cs_sum = yz / jnp.maximum(zn, 1e-30)
    npatch = float(k + 1)
    term = 200.0 * (-(cs_sum / npatch)) + tsum / npatch
    part = jnp.sum(term)

    @pl.when(q_i == 0)
    def _():
        out_ref[...] = jnp.reshape(part, (1, 1, 1))

    @pl.when(q_i != 0)
    def _():
        out_ref[...] += jnp.reshape(part, (1, 1, 1))


def _run_group(x, xs3, ys3, zs3, rsq3, idx3, k, qb):
    b, n, c = x.shape
    nq = idx3.shape[0] // b
    scalar = c / (BS * EPS_TCR)
    out = pl.pallas_call(
        functools.partial(_group_kernel, k=k, scalar=scalar),
        grid=(b, nq),
        in_specs=[
            pl.BlockSpec((1, n, c), lambda bi, qi: (bi, 0, 0)),
            pl.BlockSpec((1, 1, n), lambda bi, qi: (bi, 0, 0)),
            pl.BlockSpec((1, 1, n), lambda bi, qi: (bi, 0, 0)),
            pl.BlockSpec((1, 1, n), lambda bi, qi: (bi, 0, 0)),
            pl.BlockSpec((1, 1, n), lambda bi, qi: (bi, 0, 0)),
            pl.BlockSpec((1, qb, 1), lambda bi, qi, nq=nq: (bi * nq + qi, 0, 0)),
        ],
        out_specs=pl.BlockSpec((1, 1, 1), lambda bi, qi: (bi, 0, 0)),
        out_shape=jax.ShapeDtypeStruct((b, 1, 1), jnp.float32),
        compiler_params=pltpu.CompilerParams(
            dimension_semantics=("parallel", "arbitrary")),
    )(x, xs3, ys3, zs3, rsq3, idx3)
    return out[:, 0, 0]


def kernel(logits, logits1, p0first, p0sec, orixyz):
    b, n, c = logits.shape
    downsample = n // 16
    qb = 128

    cos_sum, rsq0, rsq1 = _run_stats(logits, logits1)
    globalpointloss = -(cos_sum / (n * b))

    p0f = jax.lax.stop_gradient(p0first)
    p0s = jax.lax.stop_gradient(p0sec)
    xs = jnp.concatenate([p0f[:, :, 0], p0s[:, :, 0]], axis=0)
    ys = jnp.concatenate([p0f[:, :, 1], p0s[:, :, 1]], axis=0)
    zs = jnp.concatenate([p0f[:, :, 2], p0s[:, :, 2]], axis=0)

    idx_all = _run_fps(xs, ys, zs, downsample)
    nq = downsample // qb
    idx0 = idx_all[:b].reshape(b * nq, qb, 1)
    idx1 = idx_all[b:].reshape(b * nq, qb, 1)

    t0 = _run_group(logits, xs[:b, None, :], ys[:b, None, :], zs[:b, None, :],
                    rsq0.reshape(b, 1, n), idx0, NSAMPLE, qb)
    t1 = _run_group(logits1, xs[b:, None, :], ys[b:, None, :], zs[b:, None, :],
                    rsq1.reshape(b, 1, n), idx1, NSAMPLE, qb)

    g = jnp.float32(0.0)
    for index in range(b):
        g = (g + t0[index] + t1[index]) / downsample
    groupinnerloss = g / b
    return globalpointloss + groupinnerloss + groupinnerloss
